# baseline (speedup 1.0000x reference)
import functools

import jax
import jax.numpy as jnp
from jax import lax
from jax.experimental import pallas as pl
from jax.experimental.pallas import tpu as pltpu
from jax.experimental.pallas import tpu_sc as plsc

E = 8
N_GROUP = 4
D = 1024
FF = 512
FFS = 2048
SCALE = 2.5
T = 2048

BM = 256
NB = T * 2 // BM + E
P = NB * BM
BTS = 512

NC, NS = 2, 16
NW = NC * NS
CT = T // NW


def _router_kernel(h_ref, rw_ref, pos1_ref, pos2_ref, w1_ref, w2_ref, be_ref,
                   oh1_s, oh2_s, cum1_s, cum2_s):
    h = h_ref[...]
    rw = rw_ref[...]
    logits = jax.lax.dot_general(
        h, rw, (((1,), (1,)), ((), ())),
        preferred_element_type=jnp.float32,
    )
    s = jax.nn.sigmoid(logits)
    r8 = jax.lax.broadcasted_iota(jnp.int32, (E, E), 0)
    c8 = jax.lax.broadcasted_iota(jnp.int32, (E, E), 1)
    perm = ((r8 ^ 1) == c8).astype(jnp.float32)
    sp = jax.lax.dot_general(
        s, perm, (((1,), (0,)), ((), ())),
        preferred_element_type=jnp.float32,
        precision=jax.lax.Precision.HIGHEST,
    )
    gs = s + sp
    eidx = jax.lax.broadcasted_iota(jnp.int32, (T, E), 1)
    gidx = eidx // 2
    m1 = jnp.max(gs, axis=1, keepdims=True)
    g1 = jnp.min(jnp.where(gs == m1, gidx, N_GROUP), axis=1, keepdims=True)
    gs2 = jnp.where(gidx == g1, -jnp.inf, gs)
    m2 = jnp.max(gs2, axis=1, keepdims=True)
    g2 = jnp.min(jnp.where(gs2 == m2, gidx, N_GROUP), axis=1, keepdims=True)
    sel = (gidx == g1) | (gidx == g2)
    masked = jnp.where(sel, s, 0.0)
    w1 = jnp.max(masked, axis=1, keepdims=True)
    e1 = jnp.min(jnp.where(masked == w1, eidx, E), axis=1, keepdims=True)
    masked2 = jnp.where(eidx == e1, -1.0, masked)
    w2 = jnp.max(masked2, axis=1, keepdims=True)
    e2 = jnp.min(jnp.where(masked2 == w2, eidx, E), axis=1, keepdims=True)
    denom = w1 + w2 + 1e-20
    w1_ref[...] = jnp.broadcast_to(w1 * (SCALE / denom), (T, 16))
    w2_ref[...] = jnp.broadcast_to(w2 * (SCALE / denom), (T, 16))

    oh1_s[...] = (eidx == e1).astype(jnp.float32)
    oh2_s[...] = (eidx == e2).astype(jnp.float32)
    rl = jax.lax.broadcasted_iota(jnp.int32, (BM, BM), 0)
    cl = jax.lax.broadcasted_iota(jnp.int32, (BM, BM), 1)
    ltri = (rl > cl).astype(jnp.float32)

    def chunk_cumsum(i, carry, oh_s, cum_s):
        sl = pl.ds(i * BM, BM)
        blk = oh_s[sl, :]
        local = jax.lax.dot_general(ltri, blk, (((1,), (0,)), ((), ())),
                                    preferred_element_type=jnp.float32)
        cum_s[sl, :] = local + carry
        return carry + jnp.sum(blk, axis=0, keepdims=True)

    tot1 = jax.lax.fori_loop(
        0, T // BM, lambda i, c: chunk_cumsum(i, c, oh1_s, cum1_s),
        jnp.zeros((1, E), jnp.float32))
    tot2 = jax.lax.fori_loop(
        0, T // BM, lambda i, c: chunk_cumsum(i, c, oh2_s, cum2_s),
        jnp.zeros((1, E), jnp.float32))

    cnt = (tot1 + tot2).astype(jnp.int32)
    padded = ((cnt + (BM - 1)) // BM) * BM
    tri8 = (r8 < c8).astype(jnp.float32)
    base = jax.lax.dot_general(padded.astype(jnp.float32), tri8,
                               (((1,), (0,)), ((), ())),
                               preferred_element_type=jnp.float32,
                               precision=jax.lax.Precision.HIGHEST)
    pos1 = jnp.sum(oh1_s[...] * (cum1_s[...] + base), axis=1, keepdims=True)
    pos2 = jnp.sum(oh2_s[...] * (cum2_s[...] + base + tot1), axis=1,
                   keepdims=True)
    pos1_ref[...] = pos1.astype(jnp.int32)
    pos2_ref[...] = pos2.astype(jnp.int32)

    cp = base + padded.astype(jnp.float32)
    eye8 = (r8 == c8).astype(jnp.float32)
    cp_sub = jax.lax.dot_general(eye8, cp, (((1,), (1,)), ((), ())),
                                 preferred_element_type=jnp.float32,
                                 precision=jax.lax.Precision.HIGHEST)
    blk = jax.lax.broadcasted_iota(jnp.int32, (E, NB), 1).astype(jnp.float32)
    a = (cp_sub <= blk * BM).astype(jnp.float32)
    be = jnp.sum(a, axis=0, keepdims=True)
    be_ref[...] = jnp.minimum(be, E - 1).astype(jnp.int32)


def _dispatch_body(h_hbm, pos1_hbm, pos2_hbm, hg_hbm,
                   idx1_v, idx2_v, rows_v, sem):
    wid = lax.axis_index("s") * NC + lax.axis_index("c")
    base = wid * CT
    c0 = pltpu.async_copy(h_hbm.at[pl.ds(base, CT)], rows_v, sem)
    pltpu.sync_copy(pos1_hbm.at[pl.ds(base, CT)], idx1_v)
    pltpu.sync_copy(pos2_hbm.at[pl.ds(base, CT)], idx2_v)
    c0.wait()
    c3 = pltpu.async_copy(rows_v, hg_hbm.at[idx1_v], sem)
    c4 = pltpu.async_copy(rows_v, hg_hbm.at[idx2_v], sem)
    c3.wait()
    c4.wait()


CC = 16


def _make_combine(TH):
    CTH = TH // NW
    NCHH = CTH // CC

    def body(y_hbm, sh_hbm, pos1_hbm, pos2_hbm, w1_hbm, w2_hbm, out_hbm,
             idx1_v, idx2_v, y1_v, y2_v, sh_v, w1b_v, w2b_v, sem):
        wid = lax.axis_index("s") * NC + lax.axis_index("c")

        def issue(c):
            base = wid * CTH + c * CC
            b = c % 2
            pltpu.sync_copy(pos1_hbm.at[pl.ds(base, CC)], idx1_v.at[b])
            pltpu.sync_copy(pos2_hbm.at[pl.ds(base, CC)], idx2_v.at[b])
            pltpu.sync_copy(w1_hbm.at[pl.ds(base, CC)], w1b_v.at[b])
            pltpu.sync_copy(w2_hbm.at[pl.ds(base, CC)], w2b_v.at[b])
            g1 = pltpu.async_copy(y_hbm.at[idx1_v.at[b]], y1_v.at[b], sem)
            g2 = pltpu.async_copy(y_hbm.at[idx2_v.at[b]], y2_v.at[b], sem)
            g3 = pltpu.async_copy(sh_hbm.at[pl.ds(base, CC)], sh_v.at[b], sem)
            return (g1, g2, g3)

        pend = issue(0)
        out_pend = {}
        for c in range(NCHH):
            b = c % 2
            for g in pend:
                g.wait()
            if c + 1 < NCHH:
                nb = (c + 1) % 2
                if nb in out_pend:
                    out_pend.pop(nb).wait()
                nxt = issue(c + 1)
            else:
                nxt = ()

            def tok_body(j, _):
                wa = w1b_v[b, j, :]
                wb = w2b_v[b, j, :]
                for d in range(D // 16):
                    sl = pl.ds(d * 16, 16)
                    sh_v[b, j, sl] = (y1_v[b, j, sl] * wa + y2_v[b, j, sl] * wb
                                      + sh_v[b, j, sl])
                return 0

            jax.lax.fori_loop(0, CC, tok_body, 0)
            if b in out_pend:
                out_pend[b].wait()
            base = wid * CTH + c * CC
            out_pend[b] = pltpu.async_copy(
                sh_v.at[b], out_hbm.at[pl.ds(base, CC)], sem)
            pend = nxt
        for w in out_pend.values():
            w.wait()

    return pl.kernel(
        body, mesh=_sc_mesh,
        out_type=jax.ShapeDtypeStruct((TH, D), jnp.float32),
        scratch_types=[
            pltpu.VMEM((2, CC), jnp.int32),
            pltpu.VMEM((2, CC), jnp.int32),
            pltpu.VMEM((2, CC, D), jnp.float32),
            pltpu.VMEM((2, CC, D), jnp.float32),
            pltpu.VMEM((2, CC, D), jnp.float32),
            pltpu.VMEM((2, CC, 16), jnp.float32),
            pltpu.VMEM((2, CC, 16), jnp.float32),
            pltpu.SemaphoreType.DMA,
        ],
    )


def _gmm_kernel(be_ref, hg_ref, up_ref, down_ref, y_ref):
    hg = hg_ref[...]
    up = up_ref[0]
    dn = down_ref[0]
    a = jax.lax.dot_general(hg, up, (((1,), (1,)), ((), ())),
                            preferred_element_type=jnp.float32)
    a = a * jax.nn.sigmoid(a)
    y_ref[...] = jax.lax.dot_general(a, dn, (((1,), (1,)), ((), ())),
                                     preferred_element_type=jnp.float32)


def _shared_kernel(h_ref, sup_ref, sdn_ref, out_ref):
    h = h_ref[...]
    a = jax.lax.dot_general(h, sup_ref[...], (((1,), (1,)), ((), ())),
                            preferred_element_type=jnp.float32)
    a = a * jax.nn.sigmoid(a)
    out_ref[...] = jax.lax.dot_general(a, sdn_ref[...], (((1,), (1,)), ((), ())),
                                       preferred_element_type=jnp.float32)


_sc_mesh = plsc.VectorSubcoreMesh(core_axis_name="c", subcore_axis_name="s",
                                  num_cores=NC, num_subcores=NS)

_dispatch = pl.kernel(
    _dispatch_body, mesh=_sc_mesh,
    out_type=jax.ShapeDtypeStruct((P, D), jnp.float32),
    scratch_types=[
        pltpu.VMEM((CT,), jnp.int32),
        pltpu.VMEM((CT,), jnp.int32),
        pltpu.VMEM((CT, D), jnp.float32),
        pltpu.SemaphoreType.DMA,
    ],
)

_combine_half = _make_combine(T // 2)


def _run_router(h, router_w):
    return pl.pallas_call(
        _router_kernel,
        out_shape=(
            jax.ShapeDtypeStruct((T, 1), jnp.int32),
            jax.ShapeDtypeStruct((T, 1), jnp.int32),
            jax.ShapeDtypeStruct((T, 16), jnp.float32),
            jax.ShapeDtypeStruct((T, 16), jnp.float32),
            jax.ShapeDtypeStruct((1, NB), jnp.int32),
        ),
        in_specs=[
            pl.BlockSpec((T, D), lambda: (0, 0)),
            pl.BlockSpec((E, D), lambda: (0, 0)),
        ],
        out_specs=(
            pl.BlockSpec((T, 1), lambda: (0, 0)),
            pl.BlockSpec((T, 1), lambda: (0, 0)),
            pl.BlockSpec((T, 16), lambda: (0, 0)),
            pl.BlockSpec((T, 16), lambda: (0, 0)),
            pl.BlockSpec((1, NB), lambda: (0, 0)),
        ),
        scratch_shapes=[
            pltpu.VMEM((T, E), jnp.float32),
            pltpu.VMEM((T, E), jnp.float32),
            pltpu.VMEM((T, E), jnp.float32),
            pltpu.VMEM((T, E), jnp.float32),
        ],
        interpret=False,
    )(h, router_w)


def _run_shared(h, shared_up_w, shared_down_w):
    th = h.shape[0]
    return pl.pallas_call(
        _shared_kernel,
        grid=(th // BTS,),
        out_shape=jax.ShapeDtypeStruct((th, D), jnp.float32),
        in_specs=[
            pl.BlockSpec((BTS, D), lambda t: (t, 0)),
            pl.BlockSpec((FFS, D), lambda t: (0, 0)),
            pl.BlockSpec((D, FFS), lambda t: (0, 0)),
        ],
        out_specs=pl.BlockSpec((BTS, D), lambda t: (t, 0)),
        compiler_params=pltpu.CompilerParams(
            dimension_semantics=("arbitrary",),
        ),
        interpret=False,
    )(h, shared_up_w, shared_down_w)


def _run_gmm(be, hg, up_w, down_w):
    return pl.pallas_call(
        _gmm_kernel,
        grid_spec=pltpu.PrefetchScalarGridSpec(
            num_scalar_prefetch=1,
            grid=(NB,),
            in_specs=[
                pl.BlockSpec((BM, D), lambda b, be_r: (b, 0)),
                pl.BlockSpec((1, FF, D), lambda b, be_r: (be_r[b], 0, 0)),
                pl.BlockSpec((1, D, FF), lambda b, be_r: (be_r[b], 0, 0)),
            ],
            out_specs=pl.BlockSpec((BM, D), lambda b, be_r: (b, 0)),
        ),
        out_shape=jax.ShapeDtypeStruct((P, D), jnp.float32),
        compiler_params=pltpu.CompilerParams(
            dimension_semantics=("arbitrary",),
        ),
        interpret=False,
    )(be.reshape(NB), hg, up_w, down_w)


def kernel(hidden_states, router_w, up_w, down_w, shared_up_w, shared_down_w):
    orig_shape = hidden_states.shape
    h = hidden_states.reshape(T, D)
    pos1, pos2, w1, w2, be = _run_router(h, router_w)
    pos1f = pos1.reshape(T)
    pos2f = pos2.reshape(T)
    t2 = T // 2
    sh1 = _run_shared(h[:t2], shared_up_w, shared_down_w)
    hg = _dispatch(h, pos1f, pos2f)
    y = _run_gmm(be, hg, up_w, down_w)
    h2, yb = lax.optimization_barrier((h[t2:], y))
    sh2 = _run_shared(h2, shared_up_w, shared_down_w)
    o1 = _combine_half(yb, sh1, pos1f[:t2], pos2f[:t2], w1[:t2], w2[:t2])
    o2 = _combine_half(yb, sh2, pos1f[t2:], pos2f[t2:], w1[t2:], w2[t2:])
    out = jnp.concatenate([o1, o2], axis=0)
    return out.reshape(orig_shape)


# speedup vs baseline: 1.1456x; 1.1456x over previous
import functools

import jax
import jax.numpy as jnp
from jax import lax
from jax.experimental import pallas as pl
from jax.experimental.pallas import tpu as pltpu
from jax.experimental.pallas import tpu_sc as plsc

E = 8
N_GROUP = 4
D = 1024
FF = 512
FFS = 2048
SCALE = 2.5
T = 2048

BM = 256
NB = T * 2 // BM + E
P = NB * BM
BTS = 512

NC, NS = 2, 16
NW = NC * NS
CT = T // NW


def _router_kernel(h_ref, rw_ref, pos1_ref, pos2_ref, w1_ref, w2_ref, be_ref,
                   oh1_s, oh2_s, cum1_s, cum2_s):
    h = h_ref[...]
    rw = rw_ref[...]
    logits = jax.lax.dot_general(
        h, rw, (((1,), (1,)), ((), ())),
        preferred_element_type=jnp.float32,
    )
    s = jax.nn.sigmoid(logits)
    r8 = jax.lax.broadcasted_iota(jnp.int32, (E, E), 0)
    c8 = jax.lax.broadcasted_iota(jnp.int32, (E, E), 1)
    perm = ((r8 ^ 1) == c8).astype(jnp.float32)
    sp = jax.lax.dot_general(
        s, perm, (((1,), (0,)), ((), ())),
        preferred_element_type=jnp.float32,
        precision=jax.lax.Precision.HIGHEST,
    )
    gs = s + sp
    eidx = jax.lax.broadcasted_iota(jnp.int32, (T, E), 1)
    gidx = eidx // 2
    m1 = jnp.max(gs, axis=1, keepdims=True)
    g1 = jnp.min(jnp.where(gs == m1, gidx, N_GROUP), axis=1, keepdims=True)
    gs2 = jnp.where(gidx == g1, -jnp.inf, gs)
    m2 = jnp.max(gs2, axis=1, keepdims=True)
    g2 = jnp.min(jnp.where(gs2 == m2, gidx, N_GROUP), axis=1, keepdims=True)
    sel = (gidx == g1) | (gidx == g2)
    masked = jnp.where(sel, s, 0.0)
    w1 = jnp.max(masked, axis=1, keepdims=True)
    e1 = jnp.min(jnp.where(masked == w1, eidx, E), axis=1, keepdims=True)
    masked2 = jnp.where(eidx == e1, -1.0, masked)
    w2 = jnp.max(masked2, axis=1, keepdims=True)
    e2 = jnp.min(jnp.where(masked2 == w2, eidx, E), axis=1, keepdims=True)
    denom = w1 + w2 + 1e-20
    w1_ref[...] = jnp.broadcast_to(w1 * (SCALE / denom), (T, 16))
    w2_ref[...] = jnp.broadcast_to(w2 * (SCALE / denom), (T, 16))

    oh1_s[...] = (eidx == e1).astype(jnp.float32)
    oh2_s[...] = (eidx == e2).astype(jnp.float32)
    rl = jax.lax.broadcasted_iota(jnp.int32, (BM, BM), 0)
    cl = jax.lax.broadcasted_iota(jnp.int32, (BM, BM), 1)
    ltri = (rl > cl).astype(jnp.float32)

    def chunk_cumsum(i, carry, oh_s, cum_s):
        sl = pl.ds(i * BM, BM)
        blk = oh_s[sl, :]
        local = jax.lax.dot_general(ltri, blk, (((1,), (0,)), ((), ())),
                                    preferred_element_type=jnp.float32)
        cum_s[sl, :] = local + carry
        return carry + jnp.sum(blk, axis=0, keepdims=True)

    tot1 = jax.lax.fori_loop(
        0, T // BM, lambda i, c: chunk_cumsum(i, c, oh1_s, cum1_s),
        jnp.zeros((1, E), jnp.float32))
    tot2 = jax.lax.fori_loop(
        0, T // BM, lambda i, c: chunk_cumsum(i, c, oh2_s, cum2_s),
        jnp.zeros((1, E), jnp.float32))

    cnt = (tot1 + tot2).astype(jnp.int32)
    padded = ((cnt + (BM - 1)) // BM) * BM
    tri8 = (r8 < c8).astype(jnp.float32)
    base = jax.lax.dot_general(padded.astype(jnp.float32), tri8,
                               (((1,), (0,)), ((), ())),
                               preferred_element_type=jnp.float32,
                               precision=jax.lax.Precision.HIGHEST)
    pos1 = jnp.sum(oh1_s[...] * (cum1_s[...] + base), axis=1, keepdims=True)
    pos2 = jnp.sum(oh2_s[...] * (cum2_s[...] + base + tot1), axis=1,
                   keepdims=True)
    pos1_ref[...] = pos1.astype(jnp.int32)
    pos2_ref[...] = pos2.astype(jnp.int32)

    cp = base + padded.astype(jnp.float32)
    eye8 = (r8 == c8).astype(jnp.float32)
    cp_sub = jax.lax.dot_general(eye8, cp, (((1,), (1,)), ((), ())),
                                 preferred_element_type=jnp.float32,
                                 precision=jax.lax.Precision.HIGHEST)
    blk = jax.lax.broadcasted_iota(jnp.int32, (E, NB), 1).astype(jnp.float32)
    a = (cp_sub <= blk * BM).astype(jnp.float32)
    be = jnp.sum(a, axis=0, keepdims=True)
    be_ref[...] = jnp.minimum(be, E - 1).astype(jnp.int32)


def _dispatch_body(h_hbm, pos1_hbm, pos2_hbm, hg_hbm,
                   idx1_v, idx2_v, rows_v, sem):
    wid = lax.axis_index("s") * NC + lax.axis_index("c")
    base = wid * CT
    c0 = pltpu.async_copy(h_hbm.at[pl.ds(base, CT)], rows_v, sem)
    pltpu.sync_copy(pos1_hbm.at[pl.ds(base, CT)], idx1_v)
    pltpu.sync_copy(pos2_hbm.at[pl.ds(base, CT)], idx2_v)
    c0.wait()
    c3 = pltpu.async_copy(rows_v, hg_hbm.at[idx1_v], sem)
    c4 = pltpu.async_copy(rows_v, hg_hbm.at[idx2_v], sem)
    c3.wait()
    c4.wait()


CC = 16


def _make_combine(TH):
    CTH = TH // NW
    NCHH = CTH // CC

    def body(y_hbm, sh_hbm, pos1_hbm, pos2_hbm, w1_hbm, w2_hbm, out_hbm,
             idx1_v, idx2_v, y1_v, y2_v, sh_v, w1b_v, w2b_v, sem):
        wid = lax.axis_index("s") * NC + lax.axis_index("c")

        def issue(c):
            base = wid * CTH + c * CC
            b = c % 2
            pltpu.sync_copy(pos1_hbm.at[pl.ds(base, CC)], idx1_v.at[b])
            pltpu.sync_copy(pos2_hbm.at[pl.ds(base, CC)], idx2_v.at[b])
            pltpu.sync_copy(w1_hbm.at[pl.ds(base, CC)], w1b_v.at[b])
            pltpu.sync_copy(w2_hbm.at[pl.ds(base, CC)], w2b_v.at[b])
            g1 = pltpu.async_copy(y_hbm.at[idx1_v.at[b]], y1_v.at[b], sem)
            g2 = pltpu.async_copy(y_hbm.at[idx2_v.at[b]], y2_v.at[b], sem)
            g3 = pltpu.async_copy(sh_hbm.at[pl.ds(base, CC)], sh_v.at[b], sem)
            return (g1, g2, g3)

        pend = issue(0)
        out_pend = {}
        for c in range(NCHH):
            b = c % 2
            for g in pend:
                g.wait()
            if c + 1 < NCHH:
                nb = (c + 1) % 2
                if nb in out_pend:
                    out_pend.pop(nb).wait()
                nxt = issue(c + 1)
            else:
                nxt = ()

            def tok_body(j, _):
                wa = w1b_v[b, j, :]
                wb = w2b_v[b, j, :]
                for d in range(D // 16):
                    sl = pl.ds(d * 16, 16)
                    sh_v[b, j, sl] = (y1_v[b, j, sl] * wa + y2_v[b, j, sl] * wb
                                      + sh_v[b, j, sl])
                return 0

            jax.lax.fori_loop(0, CC, tok_body, 0)
            if b in out_pend:
                out_pend[b].wait()
            base = wid * CTH + c * CC
            out_pend[b] = pltpu.async_copy(
                sh_v.at[b], out_hbm.at[pl.ds(base, CC)], sem)
            pend = nxt
        for w in out_pend.values():
            w.wait()

    return pl.kernel(
        body, mesh=_sc_mesh,
        out_type=jax.ShapeDtypeStruct((TH, D), jnp.float32),
        scratch_types=[
            pltpu.VMEM((2, CC), jnp.int32),
            pltpu.VMEM((2, CC), jnp.int32),
            pltpu.VMEM((2, CC, D), jnp.float32),
            pltpu.VMEM((2, CC, D), jnp.float32),
            pltpu.VMEM((2, CC, D), jnp.float32),
            pltpu.VMEM((2, CC, 16), jnp.float32),
            pltpu.VMEM((2, CC, 16), jnp.float32),
            pltpu.SemaphoreType.DMA,
        ],
    )


def _gmm_kernel(be_ref, hg_ref, up_ref, down_ref, y_ref):
    hg = hg_ref[...]
    up = up_ref[0]
    dn = down_ref[0]
    a = jax.lax.dot_general(hg, up, (((1,), (1,)), ((), ())),
                            preferred_element_type=jnp.float32)
    a = a * jax.nn.sigmoid(a)
    y_ref[...] = jax.lax.dot_general(a, dn, (((1,), (1,)), ((), ())),
                                     preferred_element_type=jnp.float32)


def _shared_kernel(h_ref, sup_ref, sdn_ref, out_ref):
    h = h_ref[...]
    a = jax.lax.dot_general(h, sup_ref[...], (((1,), (1,)), ((), ())),
                            preferred_element_type=jnp.float32)
    a = a * jax.nn.sigmoid(a)
    out_ref[...] = jax.lax.dot_general(a, sdn_ref[...], (((1,), (1,)), ((), ())),
                                       preferred_element_type=jnp.float32)


_sc_mesh = plsc.VectorSubcoreMesh(core_axis_name="c", subcore_axis_name="s",
                                  num_cores=NC, num_subcores=NS)

_dispatch = pl.kernel(
    _dispatch_body, mesh=_sc_mesh,
    out_type=jax.ShapeDtypeStruct((P, D), jnp.float32),
    scratch_types=[
        pltpu.VMEM((CT,), jnp.int32),
        pltpu.VMEM((CT,), jnp.int32),
        pltpu.VMEM((CT, D), jnp.float32),
        pltpu.SemaphoreType.DMA,
    ],
)

_combine = _make_combine(T)


def _run_router(h, router_w):
    return pl.pallas_call(
        _router_kernel,
        out_shape=(
            jax.ShapeDtypeStruct((T, 1), jnp.int32),
            jax.ShapeDtypeStruct((T, 1), jnp.int32),
            jax.ShapeDtypeStruct((T, 16), jnp.float32),
            jax.ShapeDtypeStruct((T, 16), jnp.float32),
            jax.ShapeDtypeStruct((1, NB), jnp.int32),
        ),
        in_specs=[
            pl.BlockSpec((T, D), lambda: (0, 0)),
            pl.BlockSpec((E, D), lambda: (0, 0)),
        ],
        out_specs=(
            pl.BlockSpec((T, 1), lambda: (0, 0)),
            pl.BlockSpec((T, 1), lambda: (0, 0)),
            pl.BlockSpec((T, 16), lambda: (0, 0)),
            pl.BlockSpec((T, 16), lambda: (0, 0)),
            pl.BlockSpec((1, NB), lambda: (0, 0)),
        ),
        scratch_shapes=[
            pltpu.VMEM((T, E), jnp.float32),
            pltpu.VMEM((T, E), jnp.float32),
            pltpu.VMEM((T, E), jnp.float32),
            pltpu.VMEM((T, E), jnp.float32),
        ],
        interpret=False,
    )(h, router_w)


def _run_shared(h, shared_up_w, shared_down_w):
    th = h.shape[0]
    return pl.pallas_call(
        _shared_kernel,
        grid=(th // BTS,),
        out_shape=jax.ShapeDtypeStruct((th, D), jnp.float32),
        in_specs=[
            pl.BlockSpec((BTS, D), lambda t: (t, 0)),
            pl.BlockSpec((FFS, D), lambda t: (0, 0)),
            pl.BlockSpec((D, FFS), lambda t: (0, 0)),
        ],
        out_specs=pl.BlockSpec((BTS, D), lambda t: (t, 0)),
        compiler_params=pltpu.CompilerParams(
            dimension_semantics=("arbitrary",),
        ),
        interpret=False,
    )(h, shared_up_w, shared_down_w)


def _run_gmm(be, hg, up_w, down_w):
    return pl.pallas_call(
        _gmm_kernel,
        grid_spec=pltpu.PrefetchScalarGridSpec(
            num_scalar_prefetch=1,
            grid=(NB,),
            in_specs=[
                pl.BlockSpec((BM, D), lambda b, be_r: (b, 0)),
                pl.BlockSpec((1, FF, D), lambda b, be_r: (be_r[b], 0, 0)),
                pl.BlockSpec((1, D, FF), lambda b, be_r: (be_r[b], 0, 0)),
            ],
            out_specs=pl.BlockSpec((BM, D), lambda b, be_r: (b, 0)),
        ),
        out_shape=jax.ShapeDtypeStruct((P, D), jnp.float32),
        compiler_params=pltpu.CompilerParams(
            dimension_semantics=("arbitrary",),
        ),
        interpret=False,
    )(be.reshape(NB), hg, up_w, down_w)


def kernel(hidden_states, router_w, up_w, down_w, shared_up_w, shared_down_w):
    orig_shape = hidden_states.shape
    h = hidden_states.reshape(T, D)
    pos1, pos2, w1, w2, be = _run_router(h, router_w)
    pos1f = pos1.reshape(T)
    pos2f = pos2.reshape(T)
    sh = _run_shared(h, shared_up_w, shared_down_w)
    hg = _dispatch(h, pos1f, pos2f)
    y = _run_gmm(be, hg, up_w, down_w)
    out = _combine(y, sh, pos1f, pos2f, w1, w2)
    return out.reshape(orig_shape)
